# baseline (speedup 1.0000x reference)
import jax
import jax.numpy as jnp
from jax.experimental import pallas as pl
from jax.experimental.pallas import tpu as pltpu

_SUB = 1024
_HALF = 512


def _fused_lowrank_kernel(x_hbm, win_hbm, wout_hbm, out_hbm,
                          xbuf, obuf, win_vmem, wout_vmem,
                          lsem, ssem, wsem):
    f32 = jnp.float32

    win_copy = pltpu.make_async_copy(win_hbm, win_vmem, wsem.at[0])
    wout_copy = pltpu.make_async_copy(wout_hbm, wout_vmem, wsem.at[1])
    win_copy.start()
    wout_copy.start()

    def load(row, nrows, slot, off, sem):
        return pltpu.make_async_copy(
            x_hbm.at[pl.ds(row, nrows), :],
            xbuf.at[slot, pl.ds(off, nrows), :],
            lsem.at[sem])

    def store(row, nrows, slot, off):
        return pltpu.make_async_copy(
            obuf.at[slot, pl.ds(off, nrows), :],
            out_hbm.at[pl.ds(row, nrows), :],
            ssem.at[slot])

    def compute_full(slot):
        h = jnp.dot(xbuf[slot], win_vmem[...], preferred_element_type=f32)
        obuf[slot] = jnp.dot(h, wout_vmem[...], preferred_element_type=f32)

    def compute_half(slot, off):
        h = jnp.dot(xbuf[slot, off:off + _HALF], win_vmem[...],
                    preferred_element_type=f32)
        obuf[slot, off:off + _HALF] = jnp.dot(
            h, wout_vmem[...], preferred_element_type=f32)

    load(0, _HALF, 0, 0, 0).start()
    load(_HALF, _HALF, 0, _HALF, 2).start()
    load(_SUB, _SUB, 1, 0, 1).start()
    win_copy.wait()
    wout_copy.wait()

    load(0, _HALF, 0, 0, 0).wait()
    compute_half(0, 0)
    store(0, _HALF, 0, 0).start()
    load(_HALF, _HALF, 0, _HALF, 2).wait()
    compute_half(0, _HALF)
    store(_HALF, _HALF, 0, _HALF).start()
    load(2 * _SUB, _SUB, 0, 0, 0).start()

    load(_SUB, _SUB, 1, 0, 1).wait()
    compute_full(1)
    store(_SUB, _SUB, 1, 0).start()
    load(3 * _SUB, _SUB, 1, 0, 1).start()

    load(2 * _SUB, _SUB, 0, 0, 0).wait()
    store(0, _HALF, 0, 0).wait()
    store(_HALF, _HALF, 0, _HALF).wait()
    compute_full(0)
    store(2 * _SUB, _SUB, 0, 0).start()
    load(4 * _SUB, _SUB, 0, 0, 0).start()

    def group_body(g, carry):
        for k, slot in ((0, 1), (1, 0)):
            step = 3 + 2 * g + k
            row = step * _SUB
            load(row, _SUB, slot, 0, slot).wait()
            store(row - 2 * _SUB, _SUB, slot, 0).wait()
            compute_full(slot)
            store(row, _SUB, slot, 0).start()
            load(row + 2 * _SUB, _SUB, slot, 0, slot).start()
        return carry

    jax.lax.fori_loop(0, 5, group_body, 0)

    row13 = 13 * _SUB
    load(row13, _SUB, 1, 0, 1).wait()
    store(row13 - 2 * _SUB, _SUB, 1, 0).wait()
    compute_full(1)
    store(row13, _SUB, 1, 0).start()
    load(15 * _SUB, _HALF, 1, 0, 1).start()
    load(15 * _SUB + _HALF, _HALF, 1, _HALF, 3).start()

    row14 = 14 * _SUB
    load(row14, _SUB, 0, 0, 0).wait()
    store(row14 - 2 * _SUB, _SUB, 0, 0).wait()
    compute_full(0)
    store(row14, _SUB, 0, 0).start()

    store(row13, _SUB, 1, 0).wait()
    load(15 * _SUB, _HALF, 1, 0, 1).wait()
    compute_half(1, 0)
    store(15 * _SUB, _HALF, 1, 0).start()
    load(15 * _SUB + _HALF, _HALF, 1, _HALF, 3).wait()
    compute_half(1, _HALF)
    store(15 * _SUB + _HALF, _HALF, 1, _HALF).start()

    store(row14, _SUB, 0, 0).wait()
    store(15 * _SUB, _HALF, 1, 0).wait()
    store(15 * _SUB + _HALF, _HALF, 1, _HALF).wait()


@jax.jit
def kernel(x, shared_in, shared_out):
    n_tokens, in_features = x.shape
    rank, out_features = shared_out.shape

    return pl.pallas_call(
        _fused_lowrank_kernel,
        in_specs=[
            pl.BlockSpec(memory_space=pltpu.MemorySpace.HBM),
            pl.BlockSpec(memory_space=pltpu.MemorySpace.HBM),
            pl.BlockSpec(memory_space=pltpu.MemorySpace.HBM),
        ],
        out_specs=pl.BlockSpec(memory_space=pltpu.MemorySpace.HBM),
        out_shape=jax.ShapeDtypeStruct((n_tokens, out_features), jnp.float32),
        scratch_shapes=[
            pltpu.VMEM((2, _SUB, in_features), jnp.float32),
            pltpu.VMEM((2, _SUB, out_features), jnp.float32),
            pltpu.VMEM((in_features, rank), jnp.float32),
            pltpu.VMEM((rank, out_features), jnp.float32),
            pltpu.SemaphoreType.DMA((4,)),
            pltpu.SemaphoreType.DMA((2,)),
            pltpu.SemaphoreType.DMA((2,)),
        ],
    )(x, shared_in, shared_out)


# speedup vs baseline: 1.0129x; 1.0129x over previous
import jax
import jax.numpy as jnp
from jax.experimental import pallas as pl
from jax.experimental.pallas import tpu as pltpu

_SUB = 512
_NL = 8
_PF = 4
_KS = 4


def _fused_lowrank_kernel(x_hbm, win_hbm, wout_hbm, out_hbm,
                          xbuf, obuf, win_vmem, wout_vmem,
                          lsem, ssem, wsem):
    f32 = jnp.float32
    n_tokens = x_hbm.shape[0]
    n_steps = n_tokens // _SUB
    n_groups = n_steps // _NL

    win_copy = pltpu.make_async_copy(win_hbm, win_vmem, wsem.at[0])
    wout_copy = pltpu.make_async_copy(wout_hbm, wout_vmem, wsem.at[1])
    win_copy.start()
    wout_copy.start()

    def load(step, slot):
        return pltpu.make_async_copy(
            x_hbm.at[pl.ds(step * _SUB, _SUB), :], xbuf.at[slot],
            lsem.at[slot])

    def store(step, slot):
        return pltpu.make_async_copy(
            obuf.at[slot], out_hbm.at[pl.ds(step * _SUB, _SUB), :],
            ssem.at[slot])

    def compute(lslot, oslot):
        h = jnp.dot(xbuf[lslot], win_vmem[...], preferred_element_type=f32)
        obuf[oslot] = jnp.dot(h, wout_vmem[...], preferred_element_type=f32)

    for k in range(_PF):
        load(k, k).start()
    win_copy.wait()
    wout_copy.wait()

    for k in range(_NL):
        load(k + _PF, (k + _PF) % _NL).start()
        load(k, k).wait()
        if k >= _KS:
            store(k - _KS, k % _KS).wait()
        compute(k, k % _KS)
        store(k, k % _KS).start()

    def group_body(g, carry):
        base = g * _NL
        for k in range(_NL):
            step = base + k
            load(step + _PF, (k + _PF) % _NL).start()
            load(step, k).wait()
            store(step - _KS, k % _KS).wait()
            compute(k, k % _KS)
            store(step, k % _KS).start()
        return carry

    jax.lax.fori_loop(1, n_groups - 1, group_body, 0)

    base = (n_groups - 1) * _NL
    for k in range(_NL):
        step = base + k
        if step + _PF < n_steps:
            load(step + _PF, (k + _PF) % _NL).start()
        load(step, k).wait()
        store(step - _KS, k % _KS).wait()
        compute(k, k % _KS)
        store(step, k % _KS).start()

    for step in range(n_steps - _KS, n_steps):
        store(step, step % _KS).wait()


@jax.jit
def kernel(x, shared_in, shared_out):
    n_tokens, in_features = x.shape
    rank, out_features = shared_out.shape

    return pl.pallas_call(
        _fused_lowrank_kernel,
        in_specs=[
            pl.BlockSpec(memory_space=pltpu.MemorySpace.HBM),
            pl.BlockSpec(memory_space=pltpu.MemorySpace.HBM),
            pl.BlockSpec(memory_space=pltpu.MemorySpace.HBM),
        ],
        out_specs=pl.BlockSpec(memory_space=pltpu.MemorySpace.HBM),
        out_shape=jax.ShapeDtypeStruct((n_tokens, out_features), jnp.float32),
        scratch_shapes=[
            pltpu.VMEM((_NL, _SUB, in_features), jnp.float32),
            pltpu.VMEM((_KS, _SUB, out_features), jnp.float32),
            pltpu.VMEM((in_features, rank), jnp.float32),
            pltpu.VMEM((rank, out_features), jnp.float32),
            pltpu.SemaphoreType.DMA((_NL,)),
            pltpu.SemaphoreType.DMA((_KS,)),
            pltpu.SemaphoreType.DMA((2,)),
        ],
    )(x, shared_in, shared_out)


# speedup vs baseline: 1.0248x; 1.0118x over previous
import jax
import jax.numpy as jnp
from jax.experimental import pallas as pl
from jax.experimental.pallas import tpu as pltpu

_SUB = 512
_NL = 8
_PF = 4
_KS = 4
_EDGE = 128


def _fused_lowrank_kernel(x_hbm, win_hbm, wout_hbm, out_hbm,
                          xbuf, obuf, win_vmem, wout_vmem,
                          lsem, ssem, wsem):
    f32 = jnp.float32
    n_tokens = x_hbm.shape[0]
    n_steps = n_tokens // _SUB
    n_groups = n_steps // _NL

    win_copy = pltpu.make_async_copy(win_hbm, win_vmem, wsem.at[0])
    wout_copy = pltpu.make_async_copy(wout_hbm, wout_vmem, wsem.at[1])
    win_copy.start()
    wout_copy.start()

    def load(step, slot):
        return pltpu.make_async_copy(
            x_hbm.at[pl.ds(step * _SUB, _SUB), :], xbuf.at[slot],
            lsem.at[slot])

    def store(step, slot):
        return pltpu.make_async_copy(
            obuf.at[slot], out_hbm.at[pl.ds(step * _SUB, _SUB), :],
            ssem.at[slot])

    def load_chunk(step, slot, c, sem):
        row = step * _SUB + c * _EDGE
        return pltpu.make_async_copy(
            x_hbm.at[pl.ds(row, _EDGE), :],
            xbuf.at[slot, pl.ds(c * _EDGE, _EDGE), :],
            lsem.at[sem])

    def store_chunk(step, slot, c):
        row = step * _SUB + c * _EDGE
        return pltpu.make_async_copy(
            obuf.at[slot, pl.ds(c * _EDGE, _EDGE), :],
            out_hbm.at[pl.ds(row, _EDGE), :],
            ssem.at[slot])

    def compute(lslot, oslot):
        h = jnp.dot(xbuf[lslot], win_vmem[...], preferred_element_type=f32)
        obuf[oslot] = jnp.dot(h, wout_vmem[...], preferred_element_type=f32)

    def compute_chunk(lslot, oslot, c):
        sl = slice(c * _EDGE, (c + 1) * _EDGE)
        h = jnp.dot(xbuf[lslot, sl], win_vmem[...],
                    preferred_element_type=f32)
        obuf[oslot, sl] = jnp.dot(h, wout_vmem[...],
                                  preferred_element_type=f32)

    n_chunks = _SUB // _EDGE

    for c in range(n_chunks):
        load_chunk(0, 0, c, _NL + c if c else 0).start()
    for k in range(1, _PF):
        load(k, k).start()
    win_copy.wait()
    wout_copy.wait()

    for k in range(_NL):
        load(k + _PF, (k + _PF) % _NL).start()
        if k == 0:
            for c in range(n_chunks):
                load_chunk(0, 0, c, _NL + c if c else 0).wait()
                compute_chunk(0, 0, c)
                store_chunk(0, 0, c).start()
        else:
            load(k, k).wait()
            if k >= _KS:
                if k == _KS:
                    for c in range(n_chunks):
                        store_chunk(0, 0, c).wait()
                else:
                    store(k - _KS, k % _KS).wait()
            compute(k, k % _KS)
            store(k, k % _KS).start()

    def group_body(g, carry):
        base = g * _NL
        for k in range(_NL):
            step = base + k
            load(step + _PF, (k + _PF) % _NL).start()
            load(step, k).wait()
            store(step - _KS, k % _KS).wait()
            compute(k, k % _KS)
            store(step, k % _KS).start()
        return carry

    jax.lax.fori_loop(1, n_groups - 1, group_body, 0)

    base = (n_groups - 1) * _NL
    last = n_steps - 1
    for k in range(_NL):
        step = base + k
        if step + _PF < n_steps:
            if step + _PF == last:
                for c in range(n_chunks):
                    load_chunk(last, last % _NL, c,
                               _NL + c if c else last % _NL).start()
            else:
                load(step + _PF, (k + _PF) % _NL).start()
        if step == last:
            store(last - _KS, k % _KS).wait()
            for c in range(n_chunks):
                load_chunk(last, k, c, _NL + c if c else k).wait()
                compute_chunk(k, k % _KS, c)
                store_chunk(last, k % _KS, c).start()
        else:
            load(step, k).wait()
            store(step - _KS, k % _KS).wait()
            compute(k, k % _KS)
            store(step, k % _KS).start()

    for step in range(n_steps - _KS, n_steps - 1):
        store(step, step % _KS).wait()
    for c in range(n_chunks):
        store_chunk(last, last % _KS, c).wait()


@jax.jit
def kernel(x, shared_in, shared_out):
    n_tokens, in_features = x.shape
    rank, out_features = shared_out.shape

    return pl.pallas_call(
        _fused_lowrank_kernel,
        in_specs=[
            pl.BlockSpec(memory_space=pltpu.MemorySpace.HBM),
            pl.BlockSpec(memory_space=pltpu.MemorySpace.HBM),
            pl.BlockSpec(memory_space=pltpu.MemorySpace.HBM),
        ],
        out_specs=pl.BlockSpec(memory_space=pltpu.MemorySpace.HBM),
        out_shape=jax.ShapeDtypeStruct((n_tokens, out_features), jnp.float32),
        scratch_shapes=[
            pltpu.VMEM((_NL, _SUB, in_features), jnp.float32),
            pltpu.VMEM((_KS, _SUB, out_features), jnp.float32),
            pltpu.VMEM((in_features, rank), jnp.float32),
            pltpu.VMEM((rank, out_features), jnp.float32),
            pltpu.SemaphoreType.DMA((_NL + 4,)),
            pltpu.SemaphoreType.DMA((_KS,)),
            pltpu.SemaphoreType.DMA((2,)),
        ],
    )(x, shared_in, shared_out)
